# baseline (speedup 1.0000x reference)
import functools

import jax
import jax.numpy as jnp
from jax import lax
from jax.experimental import pallas as pl
from jax.experimental.pallas import tpu as pltpu
from jax.experimental.pallas import tpu_sc as plsc

NC, NS = 2, 16
NW = NC * NS
NBUF = 8


@functools.lru_cache(maxsize=None)
def _make_sc_gather(b_total: int, hist: int, d_model: int):
    rows_w = b_total // NW
    n_groups = rows_w // NBUF
    n_pairs = n_groups // 2
    assert b_total == NW * rows_w and n_groups == 2 * n_pairs
    mesh = plsc.VectorSubcoreMesh(core_axis_name="c", subcore_axis_name="s")

    @functools.partial(
        pl.kernel,
        out_type=jax.ShapeDtypeStruct((b_total, hist, d_model), jnp.float32),
        mesh=mesh,
        scratch_types=[
            pltpu.VMEM((rows_w, hist), jnp.int32),
            pltpu.VMEM((2, NBUF, hist, d_model), jnp.float32),
            pltpu.SemaphoreType.DMA,
            pltpu.SemaphoreType.DMA,
            pltpu.SemaphoreType.DMA,
            pltpu.SemaphoreType.DMA,
        ],
        compiler_params=pltpu.CompilerParams(use_tc_tiling_on_sc=False),
    )
    def gather_kernel(x_hbm, table_hbm, out_hbm, idx_v, rows_v, gs0, gs1,
                      os0, os1):
        wid = lax.axis_index("s") * NC + lax.axis_index("c")
        base = wid * rows_w
        pltpu.sync_copy(x_hbm.at[pl.ds(base, rows_w)], idx_v)

        def fire_gathers(g, slot, gsem):
            for b in range(NBUF):
                pltpu.async_copy(
                    table_hbm.at[idx_v.at[g * NBUF + b]],
                    rows_v.at[slot, b],
                    gsem)

        def wait_group(slot, sem):
            pltpu.make_async_copy(
                out_hbm.at[pl.ds(0, NBUF)], rows_v.at[slot], sem).wait()

        def fire_out(g, slot, osem):
            pltpu.async_copy(
                rows_v.at[slot],
                out_hbm.at[pl.ds(base + g * NBUF, NBUF)],
                osem)

        fire_gathers(0, 0, gs0)

        def pair(k, carry):
            a = 2 * k

            @pl.when(k > 0)
            def _():
                wait_group(1, os1)

            fire_gathers(a + 1, 1, gs1)
            wait_group(0, gs0)
            fire_out(a, 0, os0)
            wait_group(0, os0)

            @pl.when(k < n_pairs - 1)
            def _():
                fire_gathers(a + 2, 0, gs0)

            wait_group(1, gs1)
            fire_out(a + 1, 1, os1)
            return carry

        lax.fori_loop(0, n_pairs, pair, 0)
        wait_group(1, os1)

    return gather_kernel


def kernel(x, table):
    b, h = x.shape
    return _make_sc_gather(b, h, table.shape[1])(x.astype(jnp.int32), table)


# speedup vs baseline: 1.0010x; 1.0010x over previous
import functools

import jax
import jax.numpy as jnp
from jax import lax
from jax.experimental import pallas as pl
from jax.experimental.pallas import tpu as pltpu
from jax.experimental.pallas import tpu_sc as plsc

NC, NS = 2, 16
NW = NC * NS
NBUF = 8


@functools.lru_cache(maxsize=None)
def _make_sc_gather(b_total: int, hist: int, d_model: int):
    rows_w = b_total // NW
    n_groups = rows_w // NBUF
    n_pairs = n_groups // 2
    assert b_total == NW * rows_w and n_groups == 2 * n_pairs
    mesh = plsc.VectorSubcoreMesh(core_axis_name="c", subcore_axis_name="s")

    @functools.partial(
        pl.kernel,
        out_type=jax.ShapeDtypeStruct((b_total, hist, d_model), jnp.float32),
        mesh=mesh,
        scratch_types=[
            pltpu.VMEM((rows_w * 128,), jnp.int32),
            pltpu.VMEM((2, NBUF, hist, d_model), jnp.float32),
            pltpu.SemaphoreType.DMA,
            pltpu.SemaphoreType.DMA,
            pltpu.SemaphoreType.DMA,
            pltpu.SemaphoreType.DMA,
        ],
        compiler_params=pltpu.CompilerParams(use_tc_tiling_on_sc=False),
    )
    def gather_kernel(x_hbm, table_hbm, out_hbm, idx_v, rows_v, gs0, gs1,
                      os0, os1):
        wid = lax.axis_index("s") * NC + lax.axis_index("c")
        base = wid * rows_w
        pltpu.sync_copy(x_hbm.at[pl.ds(base * 128, rows_w * 128)], idx_v)

        def fire_gathers(g, slot, gsem):
            for b in range(NBUF):
                pltpu.async_copy(
                    table_hbm.at[idx_v.at[pl.ds((g * NBUF + b) * 128, hist)]],
                    rows_v.at[slot, b],
                    gsem)

        def wait_group(slot, sem):
            pltpu.make_async_copy(
                out_hbm.at[pl.ds(0, NBUF)], rows_v.at[slot], sem).wait()

        def fire_out(g, slot, osem):
            pltpu.async_copy(
                rows_v.at[slot],
                out_hbm.at[pl.ds(base + g * NBUF, NBUF)],
                osem)

        fire_gathers(0, 0, gs0)

        def pair(k, carry):
            a = 2 * k

            @pl.when(k > 0)
            def _():
                wait_group(1, os1)

            fire_gathers(a + 1, 1, gs1)
            wait_group(0, gs0)
            fire_out(a, 0, os0)
            wait_group(0, os0)

            @pl.when(k < n_pairs - 1)
            def _():
                fire_gathers(a + 2, 0, gs0)

            wait_group(1, gs1)
            fire_out(a + 1, 1, os1)
            return carry

        lax.fori_loop(0, n_pairs, pair, 0)
        wait_group(1, os1)

    return gather_kernel


def kernel(x, table):
    b, h = x.shape
    xp = jnp.pad(x.astype(jnp.int32), ((0, 0), (0, 128 - h))).reshape(-1)
    return _make_sc_gather(b, h, table.shape[1])(xp, table)
